# Initial kernel scaffold; baseline (speedup 1.0000x reference)
import jax
import jax.numpy as jnp
from jax.experimental import pallas as pl


def kernel(x_token, x_pool, params, edge_index_tp, edge_index_pt):
    raise NotImplementedError("write your pallas kernel here")


# speedup vs baseline: 8.1410x; 8.1410x over previous
import functools

import jax
import jax.numpy as jnp
from jax import lax
from jax.experimental import pallas as pl
from jax.experimental.pallas import tpu as pltpu
from jax.experimental.pallas import tpu_sc as plsc

N_NODE = 50000
NPAD = 50176
E_EDGE = 600000
EPAD = 602112
TRASH = N_NODE
NB = 512
CB = 128
NC, NS = 2, 16
NW = NC * NS
SW = 16


def _mm_body(x_ref, w_ref, b_ref, o_ref):
    o_ref[...] = jnp.dot(x_ref[...], w_ref[...],
                         preferred_element_type=jnp.float32) + b_ref[...]


def _mm_ln_elu_body(x_ref, w_ref, b_ref, g_ref, bb_ref, o_ref):
    y = jnp.dot(x_ref[...], w_ref[...],
                preferred_element_type=jnp.float32) + b_ref[...]
    mu = jnp.mean(y, axis=-1, keepdims=True)
    d = y - mu
    var = jnp.mean(d * d, axis=-1, keepdims=True)
    y = d * jax.lax.rsqrt(var + 1e-5) * g_ref[...] + bb_ref[...]
    o_ref[...] = jnp.where(y > 0, y, jnp.exp(jnp.minimum(y, 0.0)) - 1.0)


def _tc_mm(x, w, b, g=None, bb=None):
    m, k = x.shape
    c = w.shape[1]
    full = lambda s: pl.BlockSpec(s, lambda i: (0, 0))
    ins = [pl.BlockSpec((NB, k), lambda i: (i, 0)), full((k, c)), full((1, c))]
    args = [x, w, b.reshape(1, c)]
    body = _mm_body
    if g is not None:
        ins += [full((1, c)), full((1, c))]
        args += [g.reshape(1, c), bb.reshape(1, c)]
        body = _mm_ln_elu_body
    return pl.pallas_call(
        body, grid=(m // NB,), in_specs=ins,
        out_specs=pl.BlockSpec((NB, c), lambda i: (i, 0)),
        out_shape=jax.ShapeDtypeStruct((m, c), jnp.float32))(*args)


def _score_body(xl_ref, xr_ref, a_ref, o_ref):
    h = xl_ref[...] + xr_ref[...]
    h = jnp.where(h > 0, h, 0.2 * h)
    s = jnp.dot(h, a_ref[...], preferred_element_type=jnp.float32)
    o_ref[...] = jnp.exp(s)


def _tc_score(xls, xrd, a16):
    m, c = xls.shape
    return pl.pallas_call(
        _score_body, grid=(m // NB,),
        in_specs=[pl.BlockSpec((NB, c), lambda i: (i, 0)),
                  pl.BlockSpec((NB, c), lambda i: (i, 0)),
                  pl.BlockSpec((c, SW), lambda i: (0, 0))],
        out_specs=pl.BlockSpec((NB, SW), lambda i: (i, 0)),
        out_shape=jax.ShapeDtypeStruct((m, SW), jnp.float32))(xls, xrd, a16)


def _msg_body(xl_ref, ex_ref, dn_ref, b_ref, o_ref):
    alpha = ex_ref[...] / (dn_ref[...] + 1e-16)
    ac = jnp.dot(alpha, b_ref[...], preferred_element_type=jnp.float32)
    o_ref[...] = xl_ref[...] * ac


def _tc_msg(xls, ex, deng, b16):
    m, c = xls.shape
    return pl.pallas_call(
        _msg_body, grid=(m // NB,),
        in_specs=[pl.BlockSpec((NB, c), lambda i: (i, 0)),
                  pl.BlockSpec((NB, SW), lambda i: (i, 0)),
                  pl.BlockSpec((NB, SW), lambda i: (i, 0)),
                  pl.BlockSpec((SW, c), lambda i: (0, 0))],
        out_specs=pl.BlockSpec((NB, c), lambda i: (i, 0)),
        out_shape=jax.ShapeDtypeStruct((m, c), jnp.float32))(xls, ex, deng, b16)


def _add2_body(a_ref, b_ref, o_ref):
    o_ref[...] = a_ref[...] + b_ref[...]


def _tc_add2(a, b):
    m, c = a.shape
    bs = pl.BlockSpec((NB, c), lambda i: (i, 0))
    return pl.pallas_call(
        _add2_body, grid=(m // NB,), in_specs=[bs, bs], out_specs=bs,
        out_shape=jax.ShapeDtypeStruct((m, c), jnp.float32))(a, b)


def _comb1_body(p0_ref, p1_ref, b_ref, g_ref, bb_ref, o_ref):
    y = p0_ref[...] + p1_ref[...] + b_ref[...]
    mu = jnp.mean(y, axis=-1, keepdims=True)
    d = y - mu
    var = jnp.mean(d * d, axis=-1, keepdims=True)
    y = d * jax.lax.rsqrt(var + 1e-5) * g_ref[...] + bb_ref[...]
    o_ref[...] = jnp.where(y > 0, y, jnp.exp(jnp.minimum(y, 0.0)) - 1.0)


def _tc_comb1(p0, p1, bias, g, bb):
    m, c = p0.shape
    blk = pl.BlockSpec((NB, c), lambda i: (i, 0))
    full = pl.BlockSpec((1, c), lambda i: (0, 0))
    return pl.pallas_call(
        _comb1_body, grid=(m // NB,),
        in_specs=[blk, blk, full, full, full], out_specs=blk,
        out_shape=jax.ShapeDtypeStruct((m, c), jnp.float32))(
            p0, p1, bias.reshape(1, c), g.reshape(1, c), bb.reshape(1, c))


def _comb2_body(p0_ref, p1_ref, m_ref, b_ref, o_ref):
    y = p0_ref[...] + p1_ref[...]
    o_ref[...] = jnp.dot(y, m_ref[...],
                         preferred_element_type=jnp.float32) + b_ref[...]


def _tc_comb2(p0, p1, mhm, bias):
    m, c = p0.shape
    co = mhm.shape[1]
    return pl.pallas_call(
        _comb2_body, grid=(m // NB,),
        in_specs=[pl.BlockSpec((NB, c), lambda i: (i, 0)),
                  pl.BlockSpec((NB, c), lambda i: (i, 0)),
                  pl.BlockSpec((c, co), lambda i: (0, 0)),
                  pl.BlockSpec((1, co), lambda i: (0, 0))],
        out_specs=pl.BlockSpec((NB, co), lambda i: (i, 0)),
        out_shape=jax.ShapeDtypeStruct((m, co), jnp.float32))(
            p0, p1, mhm, bias.reshape(1, co))


def _sc_gather(table, idx):
    d = table.shape[1]
    b = idx.shape[0]
    bpw = b // NW
    nch = bpw // CB
    mesh = plsc.VectorSubcoreMesh(core_axis_name="c", subcore_axis_name="s")

    @functools.partial(
        pl.kernel, mesh=mesh,
        out_type=jax.ShapeDtypeStruct((b, d), jnp.float32),
        compiler_params=pltpu.CompilerParams(use_tc_tiling_on_sc=False),
        scratch_types=[pltpu.VMEM((CB,), jnp.int32),
                       pltpu.VMEM((CB, d), jnp.float32),
                       pltpu.SemaphoreType.DMA])
    def k(table_hbm, idx_hbm, out_hbm, idx_v, rows_v, sem):
        wid = lax.axis_index("s") * NC + lax.axis_index("c")
        base = wid * bpw

        def body(j, carry):
            off = base + j * CB
            pltpu.sync_copy(idx_hbm.at[pl.ds(off, CB)], idx_v)
            pltpu.async_copy(table_hbm.at[idx_v], rows_v, sem).wait()
            pltpu.sync_copy(rows_v, out_hbm.at[pl.ds(off, CB)])
            return carry

        lax.fori_loop(0, nch, body, 0)

    return k(table, idx)


def _sc_scatter_add(idx, vals, zeros):
    d = vals.shape[1]
    b = idx.shape[0]
    bpw = b // NW
    nch = bpw // CB
    nst = NPAD // NS
    mesh = plsc.VectorSubcoreMesh(core_axis_name="c", subcore_axis_name="s")

    @functools.partial(
        pl.kernel, mesh=mesh,
        out_type=jax.ShapeDtypeStruct((2 * NPAD, d), jnp.float32),
        compiler_params=pltpu.CompilerParams(use_tc_tiling_on_sc=False),
        scratch_types=[pltpu.VMEM((CB,), jnp.int32),
                       pltpu.VMEM((CB, d), jnp.float32),
                       pltpu.VMEM_SHARED((NPAD, d), jnp.float32)])
    def k(idx_hbm, vals_hbm, zeros_hbm, out_hbm, idx_v, vals_v, shared):
        cid = lax.axis_index("c")
        sid = lax.axis_index("s")
        wid = sid * NC + cid
        n0 = sid * nst
        pltpu.sync_copy(zeros_hbm.at[pl.ds(n0, nst)], shared.at[pl.ds(n0, nst)])
        plsc.subcore_barrier()
        base = wid * bpw

        def body(j, carry):
            off = base + j * CB
            pltpu.sync_copy(idx_hbm.at[pl.ds(off, CB)], idx_v)
            pltpu.sync_copy(vals_hbm.at[pl.ds(off, CB)], vals_v)
            pltpu.sync_copy(vals_v, shared.at[idx_v], add=True)
            return carry

        lax.fori_loop(0, nch, body, 0)
        plsc.subcore_barrier()
        pltpu.sync_copy(shared.at[pl.ds(n0, nst)],
                        out_hbm.at[pl.ds(cid * NPAD + n0, nst)])

    return k(idx, vals, zeros)


def _gatv2(x_src, x_dst, srcp, dstp, p, heads, out_ch):
    c = heads * out_ch
    xl = _tc_mm(x_src, p['Wl'], p['bl'])
    xr = _tc_mm(x_dst, p['Wr'], p['br'])
    xls = _sc_gather(xl, srcp)
    xrd = _sc_gather(xr, dstp)
    a16 = (p['att'][:, :, None] * jnp.eye(heads, SW)[:, None, :]).reshape(c, SW)
    ex = _tc_score(xls, xrd, a16)
    denp = _sc_scatter_add(dstp, ex, jnp.zeros((NPAD, SW), jnp.float32))
    den = _tc_add2(denp[:NPAD], denp[NPAD:])
    deng = _sc_gather(den, dstp)
    b16 = (jnp.eye(SW, heads)[:, :, None]
           * jnp.ones((out_ch,), jnp.float32)).reshape(SW, c)
    msg = _tc_msg(xls, ex, deng, b16)
    zc = jnp.zeros((NPAD, 32), jnp.float32)
    parts = [_sc_scatter_add(dstp, msg[:, c0:c0 + 32], zc)
             for c0 in range(0, c, 32)]
    p0 = jnp.concatenate([q[:NPAD] for q in parts], axis=1)
    p1 = jnp.concatenate([q[NPAD:] for q in parts], axis=1)
    return p0, p1


def kernel(x_token, x_pool, params, edge_index_tp, edge_index_pt):
    p = params
    xt = jnp.pad(x_token, ((0, NPAD - N_NODE), (0, 0)))
    xp = jnp.pad(x_pool, ((0, NPAD - N_NODE), (0, 0)))
    t = _tc_mm(xt, p['token_proj_W'], p['token_proj_b'],
               p['token_norm_g'], p['token_norm_b'])
    pool = _tc_mm(xp, p['pool_proj_W'], p['pool_proj_b'],
                  p['pool_norm_g'], p['pool_norm_b'])

    pad_i = lambda a, fill: jnp.concatenate(
        [a, jnp.full((EPAD - E_EDGE,), fill, jnp.int32)])
    src_tp, dst_tp = pad_i(edge_index_tp[0], 0), pad_i(edge_index_tp[1], TRASH)
    src_pt, dst_pt = pad_i(edge_index_pt[0], 0), pad_i(edge_index_pt[1], TRASH)

    q0, q1 = _gatv2(t, pool, src_tp, dst_tp, p['conv1_tp'], 4, 32)
    r0, r1 = _gatv2(pool, t, src_pt, dst_pt, p['conv1_pt'], 4, 32)
    p_hid = _tc_comb1(q0, q1, p['conv1_tp']['bias'],
                      p['hid_pool_norm_g'], p['hid_pool_norm_b'])
    t_hid = _tc_comb1(r0, r1, p['conv1_pt']['bias'],
                      p['hid_token_norm_g'], p['hid_token_norm_b'])

    u0, u1 = _gatv2(t_hid, p_hid, src_tp, dst_tp, p['conv2_tp'], 4, 128)
    v0, v1 = _gatv2(p_hid, t_hid, src_pt, dst_pt, p['conv2_pt'], 4, 128)
    mhm = jnp.tile(jnp.eye(128, dtype=jnp.float32), (4, 1)) * 0.25
    pool_out = _tc_comb2(u0, u1, mhm, p['conv2_tp']['bias'])
    token_out = _tc_comb2(v0, v1, mhm, p['conv2_pt']['bias'])
    return token_out[:N_NODE], pool_out[:N_NODE]
